# baseline (speedup 1.0000x reference)
import functools

import jax
import jax.numpy as jnp
from jax import lax
from jax.experimental import pallas as pl
from jax.experimental.pallas import tpu as pltpu
from jax.experimental.pallas import tpu_sc as plsc

NC = 2
NS = 16
NW = NC * NS
CHUNK = 128


def _round_up(a, m):
    return (a + m - 1) // m * m


@functools.partial(jax.jit, static_argnums=(3, 4, 5))
def _sc_aggregate(x, src, dst, n_rows, n_pad, n_chunks):
    D = x.shape[1]
    rows_per_tile = n_pad // NS
    zchunks = n_pad // CHUNK
    mesh = plsc.VectorSubcoreMesh(core_axis_name="c", subcore_axis_name="s")
    ph = n_chunks // 2 + 1

    @functools.partial(
        pl.kernel,
        out_type=jax.ShapeDtypeStruct((NC, n_pad, D), jnp.float32),
        mesh=mesh,
        scratch_types=[
            pltpu.VMEM_SHARED((n_pad, D), jnp.float32),
            pltpu.VMEM((ph, CHUNK), jnp.int32),
            pltpu.VMEM((ph, CHUNK), jnp.int32),
            pltpu.VMEM((CHUNK, D), jnp.float32),
            pltpu.VMEM((CHUNK, D), jnp.float32),
            pltpu.SemaphoreType.DMA,
            pltpu.SemaphoreType.DMA,
        ],
    )
    def k(x_hbm, src_hbm, dst_hbm, out_hbm, aggr_sh, src_v, dst_v,
          buf0, buf1, sem0, sem1):
        c = lax.axis_index("c")
        s = lax.axis_index("s")

        zero = jnp.zeros((16,), jnp.float32)

        def zrow(i, _):
            def zcol(j, _):
                buf0[i, pl.ds(j * 16, 16)] = zero
                return 0
            return lax.fori_loop(0, D // 16, zcol, 0)

        lax.fori_loop(0, CHUNK, zrow, 0)
        for r in range((zchunks + NS - 1) // NS):
            z = s + NS * r

            @pl.when(z < zchunks)
            def _():
                pltpu.sync_copy(buf0, aggr_sh.at[pl.ds(z * CHUNK, CHUNK)])
        plsc.subcore_barrier()

        for p in range(2):
            pltpu.sync_copy(src_hbm.at[c, s, p], src_v)
            pltpu.sync_copy(dst_hbm.at[c, s, p], dst_v)
            pltpu.async_copy(x_hbm.at[src_v.at[0]], buf0, sem0)

            def body(jj, _):
                j0 = jj * 2
                j1 = j0 + 1
                pltpu.async_copy(x_hbm.at[src_v.at[j1]], buf1, sem1)
                pltpu.make_async_copy(x_hbm.at[src_v.at[j0]], buf0,
                                      sem0).wait()
                pltpu.sync_copy(buf0, aggr_sh.at[dst_v.at[j0]], add=True)
                pltpu.async_copy(x_hbm.at[src_v.at[j0 + 2]], buf0, sem0)
                pltpu.make_async_copy(x_hbm.at[src_v.at[j1]], buf1,
                                      sem1).wait()
                pltpu.sync_copy(buf1, aggr_sh.at[dst_v.at[j1]], add=True)
                return 0

            lax.fori_loop(0, (ph - 1) // 2, body, 0)
            pltpu.make_async_copy(x_hbm.at[src_v.at[ph - 1]], buf0,
                                  sem0).wait()
        plsc.subcore_barrier()

        row0 = s * rows_per_tile
        pltpu.sync_copy(
            aggr_sh.at[pl.ds(row0, rows_per_tile)],
            out_hbm.at[c, pl.ds(row0, rows_per_tile)],
        )

    return k(x, src, dst)


def _tc_linear(x, partials, W, b, relu, n_rows, block_rows):
    D = x.shape[1]
    Wt = W[:D]
    Wb = W[D:]
    b2 = b.reshape(1, D)
    grid = n_rows // block_rows

    def body(x_ref, p_ref, wt_ref, wb_ref, b_ref, o_ref):
        agg = p_ref[0] + p_ref[1]
        acc = jnp.dot(x_ref[...], wt_ref[...],
                      preferred_element_type=jnp.float32)
        acc += jnp.dot(agg, wb_ref[...], preferred_element_type=jnp.float32)
        acc += b_ref[...]
        if relu:
            acc = jnp.maximum(acc, 0.0)
        o_ref[...] = acc

    return pl.pallas_call(
        body,
        grid=(grid,),
        in_specs=[
            pl.BlockSpec((block_rows, D), lambda i: (i, 0)),
            pl.BlockSpec((NC, block_rows, D), lambda i: (0, i, 0)),
            pl.BlockSpec((D, D), lambda i: (0, 0)),
            pl.BlockSpec((D, D), lambda i: (0, 0)),
            pl.BlockSpec((1, D), lambda i: (0, 0)),
        ],
        out_specs=pl.BlockSpec((block_rows, D), lambda i: (i, 0)),
        out_shape=jax.ShapeDtypeStruct((n_rows, D), jnp.float32),
    )(x, partials, Wt, Wb, b2)


def _prep_edges(edge_index, n_rows, n_chunks):
    e = edge_index.shape[1]
    per_w = e // NW
    ew = n_chunks * CHUNK
    src = edge_index[0].reshape(NC, NS, per_w)
    dst = edge_index[1].reshape(NC, NS, per_w)
    pad = ew - per_w
    src = jnp.pad(src, ((0, 0), (0, 0), (0, pad)))
    dst = jnp.pad(dst, ((0, 0), (0, 0), (0, pad)), constant_values=n_rows)
    src = src.reshape(NC, NS, 2, n_chunks // 2, CHUNK)
    dst = dst.reshape(NC, NS, 2, n_chunks // 2, CHUNK)
    zc = jnp.zeros((NC, NS, 2, 1, CHUNK), jnp.int32)
    return (jnp.concatenate([src, zc], axis=3),
            jnp.concatenate([dst, zc + n_rows], axis=3))


def kernel(x, edge_index0, edge_index1, W0, b0, W1, b1):
    n_rows, D = x.shape
    e = edge_index0.shape[1]
    assert e % NW == 0
    n_chunks = _round_up(_round_up(e // NW, CHUNK) // CHUNK, 2)
    n_pad = _round_up(n_rows + 1, NS * CHUNK)
    block_rows = 400
    assert n_rows % block_rows == 0

    src0, dst0 = _prep_edges(edge_index0, n_rows, n_chunks)
    src1, dst1 = _prep_edges(edge_index1, n_rows, n_chunks)

    p0 = _sc_aggregate(x, src0, dst0, n_rows, n_pad, n_chunks)
    h = _tc_linear(x, p0, W0, b0, True, n_rows, block_rows)
    p1 = _sc_aggregate(h, src1, dst1, n_rows, n_pad, n_chunks)
    out = _tc_linear(h, p1, W1, b1, False, n_rows, block_rows)
    return out


# speedup vs baseline: 2.2039x; 2.2039x over previous
import functools

import jax
import jax.numpy as jnp
from jax import lax
from jax.experimental import pallas as pl
from jax.experimental.pallas import tpu as pltpu
from jax.experimental.pallas import tpu_sc as plsc

NC = 2
NS = 16
NW = NC * NS
CHUNK = 128


def _round_up(a, m):
    return (a + m - 1) // m * m


@functools.partial(jax.jit, static_argnums=(3, 4, 5))
def _sc_aggregate(x, src, dst, n_rows, n_pad, n_chunks):
    D = x.shape[1]
    rows_per_tile = n_pad // NS
    zcopies = rows_per_tile // CHUNK
    mesh = plsc.VectorSubcoreMesh(core_axis_name="c", subcore_axis_name="s")

    @functools.partial(
        pl.kernel,
        out_type=jax.ShapeDtypeStruct((NC, n_pad, D), jnp.float32),
        mesh=mesh,
        scratch_types=[
            pltpu.VMEM_SHARED((n_pad, D), jnp.float32),
            pltpu.VMEM((n_chunks, CHUNK), jnp.int32),
            pltpu.VMEM((n_chunks, CHUNK), jnp.int32),
            pltpu.VMEM((CHUNK, D), jnp.float32),
            pltpu.SemaphoreType.DMA,
            pltpu.SemaphoreType.DMA,
        ],
    )
    def k(x_hbm, src_hbm, dst_hbm, out_hbm, aggr_sh, src_v, dst_v, buf,
          sem, sem_i):
        c = lax.axis_index("c")
        s = lax.axis_index("s")

        pltpu.async_copy(src_hbm.at[c, s], src_v, sem_i)
        pltpu.async_copy(dst_hbm.at[c, s], dst_v, sem_i)

        zero = jnp.zeros((16,), jnp.float32)

        def zrow(i, _):
            def zcol(j, _):
                buf[i, pl.ds(j * 16, 16)] = zero
                return 0
            return lax.fori_loop(0, D // 16, zcol, 0)

        lax.fori_loop(0, CHUNK, zrow, 0)
        row0 = s * rows_per_tile
        for z in range(zcopies):
            pltpu.sync_copy(buf, aggr_sh.at[pl.ds(row0 + z * CHUNK, CHUNK)])
        pltpu.make_async_copy(src_hbm.at[c, s], src_v, sem_i).wait()
        pltpu.make_async_copy(dst_hbm.at[c, s], dst_v, sem_i).wait()
        plsc.subcore_barrier()

        def body(j, _):
            pltpu.async_copy(x_hbm.at[src_v.at[j]], buf, sem).wait()
            pltpu.sync_copy(buf, aggr_sh.at[dst_v.at[j]], add=True)
            return 0

        lax.fori_loop(0, n_chunks, body, 0)
        plsc.subcore_barrier()

        pltpu.sync_copy(
            aggr_sh.at[pl.ds(row0, rows_per_tile)],
            out_hbm.at[c, pl.ds(row0, rows_per_tile)],
        )

    return k(x, src, dst)


def _tc_linear(x, partials, W, b, relu, n_rows, block_rows):
    D = x.shape[1]
    Wt = W[:D]
    Wb = W[D:]
    b2 = b.reshape(1, D)
    grid = n_rows // block_rows

    def body(x_ref, p_ref, wt_ref, wb_ref, b_ref, o_ref):
        agg = p_ref[0] + p_ref[1]
        acc = jnp.dot(x_ref[...], wt_ref[...],
                      preferred_element_type=jnp.float32)
        acc += jnp.dot(agg, wb_ref[...], preferred_element_type=jnp.float32)
        acc += b_ref[...]
        if relu:
            acc = jnp.maximum(acc, 0.0)
        o_ref[...] = acc

    return pl.pallas_call(
        body,
        grid=(grid,),
        in_specs=[
            pl.BlockSpec((block_rows, D), lambda i: (i, 0)),
            pl.BlockSpec((NC, block_rows, D), lambda i: (0, i, 0)),
            pl.BlockSpec((D, D), lambda i: (0, 0)),
            pl.BlockSpec((D, D), lambda i: (0, 0)),
            pl.BlockSpec((1, D), lambda i: (0, 0)),
        ],
        out_specs=pl.BlockSpec((block_rows, D), lambda i: (i, 0)),
        out_shape=jax.ShapeDtypeStruct((n_rows, D), jnp.float32),
    )(x, partials, Wt, Wb, b2)


def _prep_edges(edge_index, n_rows, n_chunks):
    e = edge_index.shape[1]
    per_w = e // NW
    ew = n_chunks * CHUNK
    src = edge_index[0].reshape(NC, NS, per_w)
    dst = edge_index[1].reshape(NC, NS, per_w)
    pad = ew - per_w
    src = jnp.pad(src, ((0, 0), (0, 0), (0, pad)))
    dst = jnp.pad(dst, ((0, 0), (0, 0), (0, pad)), constant_values=n_rows)
    return (src.reshape(NC, NS, n_chunks, CHUNK),
            dst.reshape(NC, NS, n_chunks, CHUNK))


def kernel(x, edge_index0, edge_index1, W0, b0, W1, b1):
    n_rows, D = x.shape
    e = edge_index0.shape[1]
    assert e % NW == 0
    n_chunks = _round_up(e // NW, CHUNK) // CHUNK
    n_pad = _round_up(n_rows + 1, NS * CHUNK)
    block_rows = 400
    assert n_rows % block_rows == 0

    src0, dst0 = _prep_edges(edge_index0, n_rows, n_chunks)
    src1, dst1 = _prep_edges(edge_index1, n_rows, n_chunks)

    p0 = _sc_aggregate(x, src0, dst0, n_rows, n_pad, n_chunks)
    h = _tc_linear(x, p0, W0, b0, True, n_rows, block_rows)
    p1 = _sc_aggregate(h, src1, dst1, n_rows, n_pad, n_chunks)
    out = _tc_linear(h, p1, W1, b1, False, n_rows, block_rows)
    return out


# speedup vs baseline: 2.2061x; 1.0010x over previous
import functools

import jax
import jax.numpy as jnp
from jax import lax
from jax.experimental import pallas as pl
from jax.experimental.pallas import tpu as pltpu
from jax.experimental.pallas import tpu_sc as plsc

NC = 2
NS = 16
NW = NC * NS
CHUNK = 128


def _round_up(a, m):
    return (a + m - 1) // m * m


@functools.partial(jax.jit, static_argnums=(3, 4, 5))
def _sc_aggregate(x, src, dst, n_rows, n_pad, n_chunks):
    D = x.shape[1]
    rows_per_tile = n_pad // NS
    zcopies = (n_pad // CHUNK + NS - 1) // NS
    mesh = plsc.VectorSubcoreMesh(core_axis_name="c", subcore_axis_name="s")

    @functools.partial(
        pl.kernel,
        out_type=jax.ShapeDtypeStruct((NC, n_pad, D), jnp.float32),
        mesh=mesh,
        scratch_types=[
            pltpu.VMEM_SHARED((n_pad, D), jnp.float32),
            pltpu.VMEM((n_chunks, CHUNK), jnp.int32),
            pltpu.VMEM((n_chunks, CHUNK), jnp.int32),
            pltpu.VMEM((CHUNK, D), jnp.float32),
            pltpu.SemaphoreType.DMA,
            pltpu.SemaphoreType.DMA,
        ],
    )
    def k(x_hbm, src_hbm, dst_hbm, out_hbm, aggr_sh, src_v, dst_v, buf,
          sem, sem_i):
        c = lax.axis_index("c")
        s = lax.axis_index("s")

        pltpu.async_copy(src_hbm.at[c, s], src_v, sem_i)
        pltpu.async_copy(dst_hbm.at[c, s], dst_v, sem_i)

        zero = jnp.zeros((16,), jnp.float32)

        def zrow(i, _):
            def zcol(j, _):
                buf[i, pl.ds(j * 16, 16)] = zero
                return 0
            return lax.fori_loop(0, D // 16, zcol, 0)

        lax.fori_loop(0, CHUNK, zrow, 0)
        row0 = s * rows_per_tile
        for r in range(zcopies):
            z = s + NS * r

            @pl.when(z < n_pad // CHUNK)
            def _():
                pltpu.sync_copy(buf, aggr_sh.at[pl.ds(z * CHUNK, CHUNK)])
        pltpu.make_async_copy(src_hbm.at[c, s], src_v, sem_i).wait()
        pltpu.make_async_copy(dst_hbm.at[c, s], dst_v, sem_i).wait()
        plsc.subcore_barrier()

        def body(j, _):
            pltpu.async_copy(x_hbm.at[src_v.at[j]], buf, sem).wait()
            pltpu.sync_copy(buf, aggr_sh.at[dst_v.at[j]], add=True)
            return 0

        lax.fori_loop(0, n_chunks, body, 0)
        plsc.subcore_barrier()

        pltpu.sync_copy(
            aggr_sh.at[pl.ds(row0, rows_per_tile)],
            out_hbm.at[c, pl.ds(row0, rows_per_tile)],
        )

    return k(x, src, dst)


def _tc_linear(x, partials, W, b, relu, n_rows, block_rows):
    D = x.shape[1]
    Wt = W[:D]
    Wb = W[D:]
    b2 = b.reshape(1, D)
    grid = n_rows // block_rows

    def body(x_ref, p_ref, wt_ref, wb_ref, b_ref, o_ref):
        agg = p_ref[0] + p_ref[1]
        acc = jnp.dot(x_ref[...], wt_ref[...],
                      preferred_element_type=jnp.float32)
        acc += jnp.dot(agg, wb_ref[...], preferred_element_type=jnp.float32)
        acc += b_ref[...]
        if relu:
            acc = jnp.maximum(acc, 0.0)
        o_ref[...] = acc

    return pl.pallas_call(
        body,
        grid=(grid,),
        in_specs=[
            pl.BlockSpec((block_rows, D), lambda i: (i, 0)),
            pl.BlockSpec((NC, block_rows, D), lambda i: (0, i, 0)),
            pl.BlockSpec((D, D), lambda i: (0, 0)),
            pl.BlockSpec((D, D), lambda i: (0, 0)),
            pl.BlockSpec((1, D), lambda i: (0, 0)),
        ],
        out_specs=pl.BlockSpec((block_rows, D), lambda i: (i, 0)),
        out_shape=jax.ShapeDtypeStruct((n_rows, D), jnp.float32),
    )(x, partials, Wt, Wb, b2)


def _prep_edges(edge_index, n_rows, n_chunks):
    e = edge_index.shape[1]
    per_w = e // NW
    ew = n_chunks * CHUNK
    src = edge_index[0].reshape(NC, NS, per_w)
    dst = edge_index[1].reshape(NC, NS, per_w)
    pad = ew - per_w
    src = jnp.pad(src, ((0, 0), (0, 0), (0, pad)))
    dst = jnp.pad(dst, ((0, 0), (0, 0), (0, pad)), constant_values=n_rows)
    return (src.reshape(NC, NS, n_chunks, CHUNK),
            dst.reshape(NC, NS, n_chunks, CHUNK))


def kernel(x, edge_index0, edge_index1, W0, b0, W1, b1):
    n_rows, D = x.shape
    e = edge_index0.shape[1]
    assert e % NW == 0
    n_chunks = _round_up(e // NW, CHUNK) // CHUNK
    n_pad = _round_up(n_rows + 1, NS * 8)
    block_rows = 400
    assert n_rows % block_rows == 0

    src0, dst0 = _prep_edges(edge_index0, n_rows, n_chunks)
    src1, dst1 = _prep_edges(edge_index1, n_rows, n_chunks)

    p0 = _sc_aggregate(x, src0, dst0, n_rows, n_pad, n_chunks)
    h = _tc_linear(x, p0, W0, b0, True, n_rows, block_rows)
    p1 = _sc_aggregate(h, src1, dst1, n_rows, n_pad, n_chunks)
    out = _tc_linear(h, p1, W1, b1, False, n_rows, block_rows)
    return out
